# baseline (speedup 1.0000x reference)
import functools

import jax
import jax.numpy as jnp
from jax import lax
from jax.experimental import pallas as pl
from jax.experimental.pallas import tpu as pltpu
from jax.experimental.pallas import tpu_sc as plsc

NFEAT = 128
NHID = 16
SHID = 32
ALPHA = 0.2
NHEADS = 4
MP = 2
B = 128
N = 128
D = NHEADS * NHID

_NC = 2
_NS = 16
_NW = _NC * _NS
_CH = 128

BT = 8


def _sc_gather(emb_table, idx_flat):
    tot = idx_flat.shape[0]
    per_w = tot // _NW
    nch = per_w // _CH
    mesh = plsc.VectorSubcoreMesh(core_axis_name="c", subcore_axis_name="s")

    @functools.partial(
        pl.kernel,
        mesh=mesh,
        out_type=jax.ShapeDtypeStruct((tot, NFEAT), jnp.float32),
        scratch_types=[
            pltpu.VMEM((_CH,), jnp.int32),
            pltpu.VMEM((_CH, NFEAT), jnp.float32),
            pltpu.SemaphoreType.DMA,
        ],
    )
    def gather_kernel(table_hbm, idx_hbm, out_hbm, idx_v, rows_v, sem):
        wid = lax.axis_index("s") * _NC + lax.axis_index("c")
        base = wid * per_w
        for c in range(nch):
            off = base + c * _CH
            pltpu.sync_copy(idx_hbm.at[pl.ds(off, _CH)], idx_v)
            pltpu.async_copy(table_hbm.at[idx_v], rows_v, sem).wait()
            pltpu.sync_copy(rows_v, out_hbm.at[pl.ds(off, _CH)])

    return gather_kernel(emb_table, idx_flat)


def _hgat_body(seq_ref, aadj_ref, adj_ref, Wg_ref, Asrc_ref, AdstT_ref,
               Ws_ref, bs_ref, q_ref, z_ref, w_ref):
    g = pl.program_id(0)
    b = pl.program_id(1)
    seq = seq_ref[0, 0]
    adj_blk = jnp.where(g == 0, aadj_ref[0], adj_ref[0])
    lanes = lax.broadcasted_iota(jnp.int32, (1, 8, 128), 2)
    wvec = jnp.zeros((1, 8, 128), jnp.float32)
    for m in range(MP):
        Wh = jnp.dot(seq, Wg_ref[m], preferred_element_type=jnp.float32)
        es = jnp.dot(Wh, Asrc_ref[m], preferred_element_type=jnp.float32)
        ed = lax.dot_general(AdstT_ref[m], Wh, (((1,), (1,)), ((), ())),
                             preferred_element_type=jnp.float32)
        adj_m = adj_blk[m]
        outs = []
        for h in range(NHEADS):
            e = es[:, h:h + 1] + ed[h:h + 1, :]
            e = jnp.where(e >= 0, e, ALPHA * e)
            e = jnp.where(adj_m > 0, e, -1e9)
            e = e - jnp.max(e, axis=1, keepdims=True)
            p = jnp.exp(e)
            att = p / jnp.sum(p, axis=1, keepdims=True)
            o = jnp.dot(att, Wh[:, h * NHID:(h + 1) * NHID],
                        preferred_element_type=jnp.float32)
            outs.append(o)
        z = jnp.concatenate(outs, axis=1)
        z = jnp.where(z > 0, z, jnp.exp(z) - 1.0)
        z_ref[0, 0, m] = z
        s = jnp.tanh(jnp.dot(z, Ws_ref[...], preferred_element_type=jnp.float32)
                     + bs_ref[...])
        wm = jnp.sum(s * q_ref[...])
        wvec = wvec + jnp.where(lanes == m, wm, 0.0)

    @pl.when(b == 0)
    def _():
        w_ref[...] = wvec

    @pl.when(b > 0)
    def _():
        w_ref[...] = w_ref[...] + wvec


def _combine_body(z_ref, w_ref, msk_ref, pW1_ref, pb1_ref, pW2_ref, pb2_ref,
                  out_ref):
    g = pl.program_id(0)
    w0 = w_ref[0, 0, 0] / (B * N)
    w1 = w_ref[0, 0, 1] / (B * N)
    mx = jnp.maximum(w0, w1)
    e0 = jnp.exp(w0 - mx)
    e1 = jnp.exp(w1 - mx)
    beta0 = e0 / (e0 + e1)
    beta1 = e1 / (e0 + e1)
    rows = []
    for bb in range(BT):
        hcomb = beta0 * z_ref[0, bb, 0] + beta1 * z_ref[0, bb, 1]
        mrow = msk_ref[0, bb]
        num = jnp.sum(hcomb * mrow, axis=0, keepdims=True)
        den = jnp.sum(mrow) + 1e-10
        r = num / den
        r = jnp.where(g == 1, 1.0 / (1.0 + jnp.exp(-r)), r)
        rows.append(r)
    R = jnp.concatenate(rows, axis=0)
    t = jnp.maximum(
        jnp.dot(R, pW1_ref[...], preferred_element_type=jnp.float32)
        + pb1_ref[...], 0.0)
    out_ref[0] = (jnp.dot(t, pW2_ref[...], preferred_element_type=jnp.float32)
                  + pb2_ref[...])


def _run_tc(seq_all, aug_adjs, adjs, Wg, Asrc, AdstT, Ws, bs2, q2, msk_all,
            pW1, pb1_2, pW2, pb2_2):
    z, w = pl.pallas_call(
        _hgat_body,
        grid=(2, B),
        in_specs=[
            pl.BlockSpec((1, 1, N, NFEAT), lambda g, b: (g, b, 0, 0)),
            pl.BlockSpec((1, MP, N, N),
                         lambda g, b: (jnp.where(g == 0, b, 0), 0, 0, 0)),
            pl.BlockSpec((1, MP, N, N),
                         lambda g, b: (jnp.where(g == 0, 0, b), 0, 0, 0)),
            pl.BlockSpec((MP, NFEAT, D), lambda g, b: (0, 0, 0)),
            pl.BlockSpec((MP, D, NHEADS), lambda g, b: (0, 0, 0)),
            pl.BlockSpec((MP, NHEADS, D), lambda g, b: (0, 0, 0)),
            pl.BlockSpec((D, SHID), lambda g, b: (0, 0)),
            pl.BlockSpec((1, SHID), lambda g, b: (0, 0)),
            pl.BlockSpec((1, SHID), lambda g, b: (0, 0)),
        ],
        out_specs=[
            pl.BlockSpec((1, 1, MP, N, D), lambda g, b: (g, b, 0, 0, 0)),
            pl.BlockSpec((1, 8, 128), lambda g, b: (g, 0, 0)),
        ],
        out_shape=[
            jax.ShapeDtypeStruct((2, B, MP, N, D), jnp.float32),
            jax.ShapeDtypeStruct((2, 8, 128), jnp.float32),
        ],
    )(seq_all, aug_adjs, adjs, Wg, Asrc, AdstT, Ws, bs2, q2)

    out = pl.pallas_call(
        _combine_body,
        grid=(2, B // BT),
        in_specs=[
            pl.BlockSpec((1, BT, MP, N, D), lambda g, i: (g, i, 0, 0, 0)),
            pl.BlockSpec((1, 8, 128), lambda g, i: (g, 0, 0)),
            pl.BlockSpec((1, BT, N, 1), lambda g, i: (g, i, 0, 0)),
            pl.BlockSpec((D, D), lambda g, i: (0, 0)),
            pl.BlockSpec((1, D), lambda g, i: (0, 0)),
            pl.BlockSpec((D, D), lambda g, i: (0, 0)),
            pl.BlockSpec((1, D), lambda g, i: (0, 0)),
        ],
        out_specs=pl.BlockSpec((1, BT, D), lambda g, i: (g, i, 0)),
        out_shape=jax.ShapeDtypeStruct((2, B, D), jnp.float32),
    )(z, w, msk_all, pW1, pb1_2, pW2, pb2_2)
    return out


def kernel(items, items_aug, adjs, aug_adjs, msk, msk_aug, emb_table, W_gat,
           a_src, a_dst, Ws, bs, q, pW1, pb1, pW2, pb2):
    idx = jnp.concatenate(
        [items_aug.reshape(-1), items.reshape(-1)]).astype(jnp.int32)
    rows = _sc_gather(emb_table, idx)
    seq_all = rows.reshape(2, B, N, NFEAT)

    Wg = jnp.transpose(W_gat, (0, 2, 1, 3)).reshape(MP, NFEAT, D)
    blk = ((jnp.arange(D)[:, None] // NHID)
           == jnp.arange(NHEADS)[None, :]).astype(jnp.float32)
    Asrc = a_src.reshape(MP, D)[:, :, None] * blk[None]
    AdstT = jnp.transpose(
        a_dst.reshape(MP, D)[:, :, None] * blk[None], (0, 2, 1))
    msk_all = jnp.stack([msk_aug, msk]).reshape(2, B, N, 1)

    out = _run_tc(seq_all, aug_adjs, adjs, Wg, Asrc, AdstT, Ws,
                  bs.reshape(1, SHID), q.reshape(1, SHID), msk_all,
                  pW1, pb1.reshape(1, D), pW2, pb2.reshape(1, D))
    return (out[0], out[1])


# speedup vs baseline: 1.3410x; 1.3410x over previous
import functools

import jax
import jax.numpy as jnp
from jax import lax
from jax.experimental import pallas as pl
from jax.experimental.pallas import tpu as pltpu
from jax.experimental.pallas import tpu_sc as plsc

NFEAT = 128
NHID = 16
SHID = 32
ALPHA = 0.2
NHEADS = 4
MP = 2
B = 128
N = 128
D = NHEADS * NHID

_NC = 2
_NS = 16
_NW = _NC * _NS
_CH = 128

BT_H = 2
BT_C = 8


def _sc_gather(emb_table, idx_flat):
    tot = idx_flat.shape[0]
    per_w = tot // _NW
    nch = per_w // _CH
    mesh = plsc.VectorSubcoreMesh(core_axis_name="c", subcore_axis_name="s")

    @functools.partial(
        pl.kernel,
        mesh=mesh,
        out_type=jax.ShapeDtypeStruct((tot, NFEAT), jnp.float32),
        scratch_types=[
            pltpu.VMEM((_CH,), jnp.int32),
            pltpu.VMEM((_CH, NFEAT), jnp.float32),
            pltpu.SemaphoreType.DMA,
        ],
    )
    def gather_kernel(table_hbm, idx_hbm, out_hbm, idx_v, rows_v, sem):
        wid = lax.axis_index("s") * _NC + lax.axis_index("c")
        base = wid * per_w
        for c in range(nch):
            off = base + c * _CH
            pltpu.sync_copy(idx_hbm.at[pl.ds(off, _CH)], idx_v)
            pltpu.async_copy(table_hbm.at[idx_v], rows_v, sem).wait()
            pltpu.sync_copy(rows_v, out_hbm.at[pl.ds(off, _CH)])

    return gather_kernel(emb_table, idx_flat)


def _hgat_body(seq_ref, adj_ref, Wg_ref, Asrc_ref, AdstT_ref, Ws_ref, bs_ref,
               q_ref, z_ref, w_ref):
    step = pl.program_id(0)
    ones_row = jnp.ones((1, 128), jnp.float32)
    ones_blk = jnp.ones((N, NHID), jnp.float32)
    wrows = []
    for bb in range(BT_H):
        seq = seq_ref[bb]
        for m in range(MP):
            Wh = jnp.dot(seq, Wg_ref[m], preferred_element_type=jnp.float32)
            es = jnp.dot(Wh, Asrc_ref[m],
                         preferred_element_type=jnp.float32)
            ed = lax.dot_general(AdstT_ref[m], Wh, (((1,), (1,)), ((), ())),
                                 preferred_element_type=jnp.float32)
            adj_m = adj_ref[bb, m]
            outs = []
            for h in range(NHEADS):
                es_b = lax.dot_general(es[:, h:h + 1], ones_row,
                                       (((1,), (0,)), ((), ())),
                                       preferred_element_type=jnp.float32)
                e = es_b + ed[h:h + 1, :]
                e = jnp.where(e >= 0, e, ALPHA * e)
                p = jnp.where(adj_m > 0, jnp.exp(e), 1e-30)
                Wext = jnp.concatenate(
                    [Wh[:, h * NHID:(h + 1) * NHID], ones_blk], axis=1)
                po = jnp.dot(p, Wext, preferred_element_type=jnp.float32)
                outs.append(po[:, :NHID] / po[:, NHID:2 * NHID])
            z = jnp.concatenate(outs, axis=1)
            z = jnp.where(z > 0, z, jnp.exp(z) - 1.0)
            z_ref[bb, m] = z
            s = jnp.tanh(jnp.dot(z, Ws_ref[...],
                                 preferred_element_type=jnp.float32)
                         + bs_ref[...])
            sq = s * q_ref[...]
            cs = jnp.dot(ones_row, sq,
                         preferred_element_type=jnp.float32)
            wrows.append((m, cs))

    for m in range(MP):
        acc = sum(cs for (mm, cs) in wrows if mm == m)
        row = jnp.concatenate([acc, jnp.zeros((1, 128 - SHID), jnp.float32)],
                              axis=1)
        blk = jnp.broadcast_to(row, (8, 128))

        @pl.when(step == 0)
        def _():
            w_ref[m] = blk

        @pl.when(step > 0)
        def _():
            w_ref[m] = w_ref[m] + blk


def _combine_body(sig, z_ref, w_ref, msk_ref, pW1_ref, pb1_ref, pW2_ref,
                  pb2_ref, out_ref):
    w0 = jnp.sum(w_ref[0, 0:1, :]) / (B * N)
    w1 = jnp.sum(w_ref[1, 0:1, :]) / (B * N)
    mx = jnp.maximum(w0, w1)
    e0 = jnp.exp(w0 - mx)
    e1 = jnp.exp(w1 - mx)
    beta0 = e0 / (e0 + e1)
    beta1 = e1 / (e0 + e1)
    rows = []
    for bb in range(BT_C):
        hcomb = beta0 * z_ref[bb, 0] + beta1 * z_ref[bb, 1]
        mrow = msk_ref[bb]
        num = jnp.sum(hcomb * mrow, axis=0, keepdims=True)
        den = jnp.sum(mrow) + 1e-10
        r = num / den
        if sig:
            r = 1.0 / (1.0 + jnp.exp(-r))
        rows.append(r)
    R = jnp.concatenate(rows, axis=0)
    t = jnp.maximum(
        jnp.dot(R, pW1_ref[...], preferred_element_type=jnp.float32)
        + pb1_ref[...], 0.0)
    out_ref[...] = (jnp.dot(t, pW2_ref[...], preferred_element_type=jnp.float32)
                    + pb2_ref[...])


def _hgat_view(seq, adj, Wg, Asrc, AdstT, Ws, bs2, q2):
    return pl.pallas_call(
        _hgat_body,
        grid=(B // BT_H,),
        in_specs=[
            pl.BlockSpec((BT_H, N, NFEAT), lambda i: (i, 0, 0)),
            pl.BlockSpec((BT_H, MP, N, N), lambda i: (i, 0, 0, 0)),
            pl.BlockSpec((MP, NFEAT, D), lambda i: (0, 0, 0)),
            pl.BlockSpec((MP, D, NHEADS), lambda i: (0, 0, 0)),
            pl.BlockSpec((MP, NHEADS, D), lambda i: (0, 0, 0)),
            pl.BlockSpec((D, SHID), lambda i: (0, 0)),
            pl.BlockSpec((1, SHID), lambda i: (0, 0)),
            pl.BlockSpec((1, SHID), lambda i: (0, 0)),
        ],
        out_specs=[
            pl.BlockSpec((BT_H, MP, N, D), lambda i: (i, 0, 0, 0)),
            pl.BlockSpec((MP, 8, 128), lambda i: (0, 0, 0)),
        ],
        out_shape=[
            jax.ShapeDtypeStruct((B, MP, N, D), jnp.float32),
            jax.ShapeDtypeStruct((MP, 8, 128), jnp.float32),
        ],
    )(seq, adj, Wg, Asrc, AdstT, Ws, bs2, q2)


def _combine_view(sig, z, w, msk, pW1, pb1_2, pW2, pb2_2):
    return pl.pallas_call(
        functools.partial(_combine_body, sig),
        grid=(B // BT_C,),
        in_specs=[
            pl.BlockSpec((BT_C, MP, N, D), lambda i: (i, 0, 0, 0)),
            pl.BlockSpec((MP, 8, 128), lambda i: (0, 0, 0)),
            pl.BlockSpec((BT_C, N, 1), lambda i: (i, 0, 0)),
            pl.BlockSpec((D, D), lambda i: (0, 0)),
            pl.BlockSpec((1, D), lambda i: (0, 0)),
            pl.BlockSpec((D, D), lambda i: (0, 0)),
            pl.BlockSpec((1, D), lambda i: (0, 0)),
        ],
        out_specs=pl.BlockSpec((BT_C, D), lambda i: (i, 0)),
        out_shape=jax.ShapeDtypeStruct((B, D), jnp.float32),
    )(z, w, msk, pW1, pb1_2, pW2, pb2_2)


def kernel(items, items_aug, adjs, aug_adjs, msk, msk_aug, emb_table, W_gat,
           a_src, a_dst, Ws, bs, q, pW1, pb1, pW2, pb2):
    seq_aug = _sc_gather(
        emb_table, items_aug.reshape(-1).astype(jnp.int32)).reshape(B, N, NFEAT)
    seq_reg = _sc_gather(
        emb_table, items.reshape(-1).astype(jnp.int32)).reshape(B, N, NFEAT)

    Wg = jnp.transpose(W_gat, (0, 2, 1, 3)).reshape(MP, NFEAT, D)
    blk = ((jnp.arange(D)[:, None] // NHID)
           == jnp.arange(NHEADS)[None, :]).astype(jnp.float32)
    Asrc = a_src.reshape(MP, D)[:, :, None] * blk[None]
    AdstT = jnp.transpose(
        a_dst.reshape(MP, D)[:, :, None] * blk[None], (0, 2, 1))
    bs2 = bs.reshape(1, SHID)
    q2 = q.reshape(1, SHID)
    pb1_2 = pb1.reshape(1, D)
    pb2_2 = pb2.reshape(1, D)

    z_aug, w_aug = _hgat_view(seq_aug, aug_adjs, Wg, Asrc, AdstT, Ws, bs2, q2)
    z_reg, w_reg = _hgat_view(seq_reg, adjs, Wg, Asrc, AdstT, Ws, bs2, q2)

    c = _combine_view(False, z_aug, w_aug, msk_aug.reshape(B, N, 1),
                      pW1, pb1_2, pW2, pb2_2)
    c0 = _combine_view(True, z_reg, w_reg, msk.reshape(B, N, 1),
                       pW1, pb1_2, pW2, pb2_2)
    return (c, c0)


# speedup vs baseline: 1.6981x; 1.2663x over previous
import functools

import jax
import jax.numpy as jnp
from jax import lax
from jax.experimental import pallas as pl
from jax.experimental.pallas import tpu as pltpu
from jax.experimental.pallas import tpu_sc as plsc

NFEAT = 128
NHID = 16
SHID = 32
ALPHA = 0.2
NHEADS = 4
MP = 2
B = 128
N = 128
D = NHEADS * NHID

_NC = 2
_NS = 16
_NW = _NC * _NS
_CH = 128

BT_H = 2
BT_C = 8


def _sc_gather(emb_table, idx_flat):
    tot = idx_flat.shape[0]
    per_w = tot // _NW
    nch = per_w // _CH
    mesh = plsc.VectorSubcoreMesh(core_axis_name="c", subcore_axis_name="s")

    @functools.partial(
        pl.kernel,
        mesh=mesh,
        out_type=jax.ShapeDtypeStruct((tot, NFEAT), jnp.float32),
        scratch_types=[
            pltpu.VMEM((_CH,), jnp.int32),
            pltpu.VMEM((_CH, NFEAT), jnp.float32),
            pltpu.SemaphoreType.DMA,
        ],
    )
    def gather_kernel(table_hbm, idx_hbm, out_hbm, idx_v, rows_v, sem):
        wid = lax.axis_index("s") * _NC + lax.axis_index("c")
        base = wid * per_w
        for c in range(nch):
            off = base + c * _CH
            pltpu.sync_copy(idx_hbm.at[pl.ds(off, _CH)], idx_v)
            pltpu.async_copy(table_hbm.at[idx_v], rows_v, sem).wait()
            pltpu.sync_copy(rows_v, out_hbm.at[pl.ds(off, _CH)])

    return gather_kernel(emb_table, idx_flat)


def _hgat_body(seq_ref, adj_ref, Wg_ref, Wsrc_ref, WdstT_ref, Ws_ref, bs_ref,
               q_ref, z_ref, w_ref):
    step = pl.program_id(0)
    ones_row = jnp.ones((1, 128), jnp.float32)
    ones_blk = jnp.ones((N, D), jnp.float32)
    lane = lax.broadcasted_iota(jnp.int32, (1, 2 * D), 1)
    hid = (lane % D) // NHID
    wrows = []
    for bb in range(BT_H):
        seq = seq_ref[bb]
        for m in range(MP):
            Wh = jnp.dot(seq, Wg_ref[m], preferred_element_type=jnp.float32)
            es = jnp.dot(seq, Wsrc_ref[m],
                         preferred_element_type=jnp.float32)
            ed = lax.dot_general(WdstT_ref[m], seq, (((1,), (1,)), ((), ())),
                                 preferred_element_type=jnp.float32)
            adj_m = adj_ref[bb, m]
            WhE = jnp.concatenate([Wh, ones_blk], axis=1)
            acc = jnp.zeros((N, 2 * D), jnp.float32)
            for h in range(NHEADS):
                e = es[:, h:h + 1] + ed[h:h + 1, :]
                e = jnp.maximum(e, ALPHA * e)
                p = jnp.exp(e) * adj_m + 1e-30
                po = jnp.dot(p, WhE, preferred_element_type=jnp.float32)
                acc = acc + jnp.where(hid == h, po, 0.0)
            z = acc[:, :D] / acc[:, D:2 * D]
            z = jnp.where(z > 0, z, jnp.exp(z) - 1.0)
            z_ref[bb, m] = z
            s = jnp.tanh(jnp.dot(z, Ws_ref[...],
                                 preferred_element_type=jnp.float32)
                         + bs_ref[...])
            sq = s * q_ref[...]
            cs = jnp.dot(ones_row, sq,
                         preferred_element_type=jnp.float32)
            wrows.append((m, cs))

    for m in range(MP):
        acc = sum(cs for (mm, cs) in wrows if mm == m)
        row = jnp.concatenate([acc, jnp.zeros((1, 128 - SHID), jnp.float32)],
                              axis=1)
        blk = jnp.broadcast_to(row, (8, 128))

        @pl.when(step == 0)
        def _():
            w_ref[m] = blk

        @pl.when(step > 0)
        def _():
            w_ref[m] = w_ref[m] + blk


def _combine_body(sig, z_ref, w_ref, msk_ref, pW1_ref, pb1_ref, pW2_ref,
                  pb2_ref, out_ref):
    w0 = jnp.sum(w_ref[0, 0:1, :]) / (B * N)
    w1 = jnp.sum(w_ref[1, 0:1, :]) / (B * N)
    mx = jnp.maximum(w0, w1)
    e0 = jnp.exp(w0 - mx)
    e1 = jnp.exp(w1 - mx)
    beta0 = e0 / (e0 + e1)
    beta1 = e1 / (e0 + e1)
    rows = []
    for bb in range(BT_C):
        hcomb = beta0 * z_ref[bb, 0] + beta1 * z_ref[bb, 1]
        mrow = msk_ref[bb]
        num = jnp.sum(hcomb * mrow, axis=0, keepdims=True)
        den = jnp.sum(mrow) + 1e-10
        r = num / den
        if sig:
            r = 1.0 / (1.0 + jnp.exp(-r))
        rows.append(r)
    R = jnp.concatenate(rows, axis=0)
    t = jnp.maximum(
        jnp.dot(R, pW1_ref[...], preferred_element_type=jnp.float32)
        + pb1_ref[...], 0.0)
    out_ref[...] = (jnp.dot(t, pW2_ref[...], preferred_element_type=jnp.float32)
                    + pb2_ref[...])


def _hgat_view(seq, adj, Wg, Wsrc, WdstT, Ws, bs2, q2):
    return pl.pallas_call(
        _hgat_body,
        grid=(B // BT_H,),
        in_specs=[
            pl.BlockSpec((BT_H, N, NFEAT), lambda i: (i, 0, 0)),
            pl.BlockSpec((BT_H, MP, N, N), lambda i: (i, 0, 0, 0)),
            pl.BlockSpec((MP, NFEAT, D), lambda i: (0, 0, 0)),
            pl.BlockSpec((MP, NFEAT, NHEADS), lambda i: (0, 0, 0)),
            pl.BlockSpec((MP, NHEADS, NFEAT), lambda i: (0, 0, 0)),
            pl.BlockSpec((D, SHID), lambda i: (0, 0)),
            pl.BlockSpec((1, SHID), lambda i: (0, 0)),
            pl.BlockSpec((1, SHID), lambda i: (0, 0)),
        ],
        out_specs=[
            pl.BlockSpec((BT_H, MP, N, D), lambda i: (i, 0, 0, 0)),
            pl.BlockSpec((MP, 8, 128), lambda i: (0, 0, 0)),
        ],
        out_shape=[
            jax.ShapeDtypeStruct((B, MP, N, D), jnp.float32),
            jax.ShapeDtypeStruct((MP, 8, 128), jnp.float32),
        ],
    )(seq, adj, Wg, Wsrc, WdstT, Ws, bs2, q2)


def _combine_view(sig, z, w, msk, pW1, pb1_2, pW2, pb2_2):
    return pl.pallas_call(
        functools.partial(_combine_body, sig),
        grid=(B // BT_C,),
        in_specs=[
            pl.BlockSpec((BT_C, MP, N, D), lambda i: (i, 0, 0, 0)),
            pl.BlockSpec((MP, 8, 128), lambda i: (0, 0, 0)),
            pl.BlockSpec((BT_C, N, 1), lambda i: (i, 0, 0)),
            pl.BlockSpec((D, D), lambda i: (0, 0)),
            pl.BlockSpec((1, D), lambda i: (0, 0)),
            pl.BlockSpec((D, D), lambda i: (0, 0)),
            pl.BlockSpec((1, D), lambda i: (0, 0)),
        ],
        out_specs=pl.BlockSpec((BT_C, D), lambda i: (i, 0)),
        out_shape=jax.ShapeDtypeStruct((B, D), jnp.float32),
    )(z, w, msk, pW1, pb1_2, pW2, pb2_2)


def kernel(items, items_aug, adjs, aug_adjs, msk, msk_aug, emb_table, W_gat,
           a_src, a_dst, Ws, bs, q, pW1, pb1, pW2, pb2):
    seq_aug = _sc_gather(
        emb_table, items_aug.reshape(-1).astype(jnp.int32)).reshape(B, N, NFEAT)
    seq_reg = _sc_gather(
        emb_table, items.reshape(-1).astype(jnp.int32)).reshape(B, N, NFEAT)

    Wg = jnp.transpose(W_gat, (0, 2, 1, 3)).reshape(MP, NFEAT, D)
    Wsrc = jnp.einsum('mhfd,mhd->mfh', W_gat, a_src)
    WdstT = jnp.einsum('mhfd,mhd->mhf', W_gat, a_dst)
    bs2 = bs.reshape(1, SHID)
    q2 = q.reshape(1, SHID)
    pb1_2 = pb1.reshape(1, D)
    pb2_2 = pb2.reshape(1, D)

    z_aug, w_aug = _hgat_view(seq_aug, aug_adjs, Wg, Wsrc, WdstT, Ws, bs2, q2)
    z_reg, w_reg = _hgat_view(seq_reg, adjs, Wg, Wsrc, WdstT, Ws, bs2, q2)

    c = _combine_view(False, z_aug, w_aug, msk_aug.reshape(B, N, 1),
                      pW1, pb1_2, pW2, pb2_2)
    c0 = _combine_view(True, z_reg, w_reg, msk.reshape(B, N, 1),
                       pW1, pb1_2, pW2, pb2_2)
    return (c, c0)
